# baseline (speedup 1.0000x reference)
import jax
import jax.numpy as jnp
from jax import lax
from jax.experimental import pallas as pl
from jax.experimental.pallas import tpu as pltpu
from jax.experimental.pallas import tpu_sc as plsc

_E_BLOCK_TC = 64000
_N_BLOCK = 4096

_NW = 32
_SC_CHUNK = 16000
_SC_CHUNKS = 16
_SC_ROWS = _SC_CHUNK * _SC_CHUNKS
_SC_UNROLL = 8


def _sc_colsum_body(x1t_hbm, out_hbm, buf0, buf1, outv, sem0, sem1):
    cid = lax.axis_index("c")
    sid = lax.axis_index("s")
    w = sid * 2 + cid

    bufs = (buf0, buf1)
    sems = (sem0, sem1)
    copies = [
        pltpu.async_copy(x1t_hbm.at[w, pl.ds(0, _SC_CHUNK)], buf0, sem0),
        None,
    ]

    zero = jnp.zeros((16,), jnp.float32)
    accs = tuple(zero for _ in range(_SC_UNROLL))
    n_inner = _SC_CHUNK // (16 * _SC_UNROLL)

    for t in range(_SC_CHUNKS):
        b = t % 2
        if t + 1 < _SC_CHUNKS:
            nb = (t + 1) % 2
            copies[nb] = pltpu.async_copy(
                x1t_hbm.at[w, pl.ds((t + 1) * _SC_CHUNK, _SC_CHUNK)],
                bufs[nb], sems[nb])
        copies[b].wait()
        buf = bufs[b]

        def body(k, acc, buf=buf):
            off = k * (16 * _SC_UNROLL)
            return tuple(acc[u] + buf[pl.ds(off + u * 16, 16)]
                         for u in range(_SC_UNROLL))

        accs = lax.fori_loop(0, n_inner, body, accs)

    total = accs[0]
    for u in range(1, _SC_UNROLL):
        total = total + accs[u]
    outv[...] = total
    pltpu.sync_copy(outv, out_hbm.at[pl.ds(w * 16, 16)])


def _sc_colsum(x1t):
    mesh = plsc.VectorSubcoreMesh(core_axis_name="c", subcore_axis_name="s",
                                  num_cores=2, num_subcores=16)
    return pl.kernel(
        _sc_colsum_body,
        out_type=jax.ShapeDtypeStruct((_NW * 16,), jnp.float32),
        mesh=mesh,
        compiler_params=pltpu.CompilerParams(use_tc_tiling_on_sc=False),
        scratch_types=[
            pltpu.VMEM((_SC_CHUNK,), jnp.float32),
            pltpu.VMEM((_SC_CHUNK,), jnp.float32),
            pltpu.VMEM((16,), jnp.float32),
            pltpu.SemaphoreType.DMA,
            pltpu.SemaphoreType.DMA,
        ],
    )(x1t)


def _colsum_body(x1t_ref, out_ref):
    i = pl.program_id(0)

    @pl.when(i == 0)
    def _init():
        out_ref[...] = jnp.zeros_like(out_ref)

    x = x1t_ref[...]
    xr = x.reshape(x.shape[0], x.shape[1] // 128, 128)
    out_ref[...] += jnp.sum(xr, axis=1)


def _dense_body(s_ref, ssc_ref, x0_ref, w0cat_ref, w1on_ref, w1tg_ref,
                p1w_ref, p1b_ref, a_ref, p2w_ref, p2b_ref, inv_e_ref,
                on_ref, pred_ref, tg_ref):
    def mm(a, b):
        return lax.dot_general(a, b, (((1,), (0,)), ((), ())),
                               preferred_element_type=jnp.float32)

    def mm_t(a, b):
        return lax.dot_general(a, b, (((1,), (1,)), ((), ())),
                               preferred_element_type=jnp.float32)

    col = (jnp.sum(s_ref[...], axis=1, keepdims=True)
           + jnp.sum(ssc_ref[...], axis=1, keepdims=True))
    m = col * inv_e_ref[0, 0]
    m_on = mm(w1on_ref[...], m)
    m_tg = mm(w1tg_ref[...], m)

    x0 = x0_ref[...]
    x0_cat_t = mm_t(w0cat_ref[...], x0)
    bn = x0.shape[0]
    h_on_t = jnp.concatenate(
        [x0_cat_t[0:32, :], jnp.broadcast_to(m_on, (32, bn))], axis=0)
    h_tg_t = jnp.concatenate(
        [x0_cat_t[32:64, :], jnp.broadcast_to(m_tg, (32, bn))], axis=0)
    on_ref[...] = h_on_t
    tg_ref[...] = h_tg_t

    z = mm(p1w_ref[...], h_on_t) + p1b_ref[...]
    a = a_ref[0, 0]
    h = jnp.where(z >= 0, z, a * z)
    pred_ref[...] = mm(p2w_ref[...], h) + p2b_ref[...]


def kernel(x_0, x_1, adjacency_0, down_laplacian, up_laplacian,
           W0_on, W1_on, W0_tg, W1_tg, p1_w, p1_b, prelu_a, p2_w, p2_b):
    n, in0 = x_0.shape
    e, h = x_1.shape

    x1t = x_1.T

    sums_sc = _sc_colsum(x1t)
    ssc = sums_sc.reshape(_NW, 16)

    tc_lanes = e - _SC_ROWS
    blk0 = _SC_ROWS // _E_BLOCK_TC
    sums = pl.pallas_call(
        _colsum_body,
        grid=(tc_lanes // _E_BLOCK_TC,),
        in_specs=[pl.BlockSpec((h, _E_BLOCK_TC), lambda i: (0, i + blk0))],
        out_specs=pl.BlockSpec((h, 128), lambda i: (0, 0)),
        out_shape=jax.ShapeDtypeStruct((h, 128), jnp.float32),
    )(x1t)

    w0cat = jnp.concatenate([W0_on, W0_tg], axis=0)
    p1b = p1_b.reshape(h, 1)
    p2b = p2_b.reshape(2 * h, 1)
    a = jnp.reshape(prelu_a, (1, 1))
    inv_e = jnp.full((1, 1), 1.0 / e, dtype=jnp.float32)

    const = lambda shape: pl.BlockSpec(shape, lambda i: tuple(0 for _ in shape))
    n_blocks = (n + _N_BLOCK - 1) // _N_BLOCK
    on_t, pred_t, tg_t = pl.pallas_call(
        _dense_body,
        grid=(n_blocks,),
        in_specs=[
            const((h, 128)),
            const((_NW, 16)),
            pl.BlockSpec((_N_BLOCK, in0), lambda i: (i, 0)),
            const((2 * h, in0)),
            const((h, h)),
            const((h, h)),
            const((h, 2 * h)),
            const((h, 1)),
            const((1, 1)),
            const((2 * h, h)),
            const((2 * h, 1)),
            const((1, 1)),
        ],
        out_specs=[
            pl.BlockSpec((2 * h, _N_BLOCK), lambda i: (0, i)),
            pl.BlockSpec((2 * h, _N_BLOCK), lambda i: (0, i)),
            pl.BlockSpec((2 * h, _N_BLOCK), lambda i: (0, i)),
        ],
        out_shape=[
            jax.ShapeDtypeStruct((2 * h, n), jnp.float32),
            jax.ShapeDtypeStruct((2 * h, n), jnp.float32),
            jax.ShapeDtypeStruct((2 * h, n), jnp.float32),
        ],
    )(sums, ssc, x_0, w0cat, W1_on, W1_tg, p1_w, p1b, a, p2_w, p2b, inv_e)

    return (on_t.T, pred_t.T, tg_t.T)


# speedup vs baseline: 36.2242x; 36.2242x over previous
import jax
import jax.numpy as jnp
from jax import lax
from jax.experimental import pallas as pl
from jax.experimental.pallas import tpu as pltpu
from jax.experimental.pallas import tpu_sc as plsc

_E_BLOCK_TC = 64000
_N_BLOCK = 4096

_NW = 32
_SC_CHUNK = 16000
_SC_CHUNKS = 16
_SC_ROWS = _SC_CHUNK * _SC_CHUNKS
_SC_UNROLL = 8


def _sc_colsum_body(x1t_hbm, out_hbm, buf0, buf1, outv, sem0, sem1):
    cid = lax.axis_index("c")
    sid = lax.axis_index("s")
    w = sid * 2 + cid

    bufs = (buf0, buf1)
    sems = (sem0, sem1)
    copies = [
        pltpu.async_copy(x1t_hbm.at[w, pl.ds(0, _SC_CHUNK)], buf0, sem0),
        None,
    ]

    zero = jnp.zeros((16,), jnp.float32)
    accs = tuple(zero for _ in range(_SC_UNROLL))
    n_inner = _SC_CHUNK // (16 * _SC_UNROLL)

    for t in range(_SC_CHUNKS):
        b = t % 2
        if t + 1 < _SC_CHUNKS:
            nb = (t + 1) % 2
            copies[nb] = pltpu.async_copy(
                x1t_hbm.at[w, pl.ds((t + 1) * _SC_CHUNK, _SC_CHUNK)],
                bufs[nb], sems[nb])
        copies[b].wait()
        buf = bufs[b]

        def body(k, acc, buf=buf):
            off = k * (16 * _SC_UNROLL)
            return tuple(acc[u] + buf[pl.ds(off + u * 16, 16)]
                         for u in range(_SC_UNROLL))

        accs = lax.fori_loop(0, n_inner, body, accs)

    total = accs[0]
    for u in range(1, _SC_UNROLL):
        total = total + accs[u]
    outv[...] = total
    pltpu.sync_copy(outv, out_hbm.at[pl.ds(w * 16, 16)])


def _sc_colsum(x1t):
    mesh = plsc.VectorSubcoreMesh(core_axis_name="c", subcore_axis_name="s",
                                  num_cores=2, num_subcores=16)
    return pl.kernel(
        _sc_colsum_body,
        out_type=jax.ShapeDtypeStruct((_NW * 16,), jnp.float32),
        mesh=mesh,
        compiler_params=pltpu.CompilerParams(use_tc_tiling_on_sc=True),
        scratch_types=[
            pltpu.VMEM((_SC_CHUNK,), jnp.float32),
            pltpu.VMEM((_SC_CHUNK,), jnp.float32),
            pltpu.VMEM((16,), jnp.float32),
            pltpu.SemaphoreType.DMA,
            pltpu.SemaphoreType.DMA,
        ],
    )(x1t)


def _colsum_body(x1t_ref, out_ref):
    i = pl.program_id(0)

    @pl.when(i == 0)
    def _init():
        out_ref[...] = jnp.zeros_like(out_ref)

    x = x1t_ref[...]
    xr = x.reshape(x.shape[0], x.shape[1] // 128, 128)
    out_ref[...] += jnp.sum(xr, axis=1)


def _dense_body(s_ref, ssc_ref, x0_ref, w0cat_ref, w1on_ref, w1tg_ref,
                p1w_ref, p1b_ref, a_ref, p2w_ref, p2b_ref, inv_e_ref,
                on_ref, pred_ref, tg_ref):
    def mm(a, b):
        return lax.dot_general(a, b, (((1,), (0,)), ((), ())),
                               preferred_element_type=jnp.float32)

    def mm_t(a, b):
        return lax.dot_general(a, b, (((1,), (1,)), ((), ())),
                               preferred_element_type=jnp.float32)

    col = (jnp.sum(s_ref[...], axis=1, keepdims=True)
           + jnp.sum(ssc_ref[...], axis=1, keepdims=True))
    m = col * inv_e_ref[0, 0]
    m_on = mm(w1on_ref[...], m)
    m_tg = mm(w1tg_ref[...], m)

    x0 = x0_ref[...]
    x0_cat_t = mm_t(w0cat_ref[...], x0)
    bn = x0.shape[0]
    h_on_t = jnp.concatenate(
        [x0_cat_t[0:32, :], jnp.broadcast_to(m_on, (32, bn))], axis=0)
    h_tg_t = jnp.concatenate(
        [x0_cat_t[32:64, :], jnp.broadcast_to(m_tg, (32, bn))], axis=0)
    on_ref[...] = h_on_t
    tg_ref[...] = h_tg_t

    z = mm(p1w_ref[...], h_on_t) + p1b_ref[...]
    a = a_ref[0, 0]
    h = jnp.where(z >= 0, z, a * z)
    pred_ref[...] = mm(p2w_ref[...], h) + p2b_ref[...]


def kernel(x_0, x_1, adjacency_0, down_laplacian, up_laplacian,
           W0_on, W1_on, W0_tg, W1_tg, p1_w, p1_b, prelu_a, p2_w, p2_b):
    n, in0 = x_0.shape
    e, h = x_1.shape

    x1t = x_1.T

    sums_sc = _sc_colsum(x1t)
    ssc = sums_sc.reshape(_NW, 16)

    tc_lanes = e - _SC_ROWS
    blk0 = _SC_ROWS // _E_BLOCK_TC
    sums = pl.pallas_call(
        _colsum_body,
        grid=(tc_lanes // _E_BLOCK_TC,),
        in_specs=[pl.BlockSpec((h, _E_BLOCK_TC), lambda i: (0, i + blk0))],
        out_specs=pl.BlockSpec((h, 128), lambda i: (0, 0)),
        out_shape=jax.ShapeDtypeStruct((h, 128), jnp.float32),
    )(x1t)

    w0cat = jnp.concatenate([W0_on, W0_tg], axis=0)
    p1b = p1_b.reshape(h, 1)
    p2b = p2_b.reshape(2 * h, 1)
    a = jnp.reshape(prelu_a, (1, 1))
    inv_e = jnp.full((1, 1), 1.0 / e, dtype=jnp.float32)

    const = lambda shape: pl.BlockSpec(shape, lambda i: tuple(0 for _ in shape))
    n_blocks = (n + _N_BLOCK - 1) // _N_BLOCK
    on_t, pred_t, tg_t = pl.pallas_call(
        _dense_body,
        grid=(n_blocks,),
        in_specs=[
            const((h, 128)),
            const((_NW, 16)),
            pl.BlockSpec((_N_BLOCK, in0), lambda i: (i, 0)),
            const((2 * h, in0)),
            const((h, h)),
            const((h, h)),
            const((h, 2 * h)),
            const((h, 1)),
            const((1, 1)),
            const((2 * h, h)),
            const((2 * h, 1)),
            const((1, 1)),
        ],
        out_specs=[
            pl.BlockSpec((2 * h, _N_BLOCK), lambda i: (0, i)),
            pl.BlockSpec((2 * h, _N_BLOCK), lambda i: (0, i)),
            pl.BlockSpec((2 * h, _N_BLOCK), lambda i: (0, i)),
        ],
        out_shape=[
            jax.ShapeDtypeStruct((2 * h, n), jnp.float32),
            jax.ShapeDtypeStruct((2 * h, n), jnp.float32),
            jax.ShapeDtypeStruct((2 * h, n), jnp.float32),
        ],
    )(sums, ssc, x_0, w0cat, W1_on, W1_tg, p1_w, p1b, a, p2_w, p2b, inv_e)

    return (on_t.T, pred_t.T, tg_t.T)
